# Initial kernel scaffold; baseline (speedup 1.0000x reference)
import jax
import jax.numpy as jnp
from jax.experimental import pallas as pl


def kernel(node_final_id, edge_index, edge_type, embedding, comp, bases, root, bias):
    raise NotImplementedError("write your pallas kernel here")


# speedup vs baseline: 32.5908x; 32.5908x over previous
import functools

import jax
import jax.numpy as jnp
from jax import lax
from jax.experimental import pallas as pl
from jax.experimental.pallas import tpu as pltpu
from jax.experimental.pallas import tpu_sc as plsc

_N = 10000
_NP = 10240
_E = 320000
_D = 128
_R = 39
_B = 8

_NT = 16
_EPT = _E // _NT
_SUBE = 4000
_NSUB = _EPT // _SUBE
_CAP = _R * _NP
_SLAB = _CAP // _NT

_XW = 25
_XPW = _N // _XW

_mesh = plsc.VectorSubcoreMesh(
    core_axis_name="c", subcore_axis_name="s", num_cores=2, num_subcores=_NT
)


@functools.partial(
    pl.kernel,
    out_type=(
        jax.ShapeDtypeStruct((_CAP,), jnp.float32),
        jax.ShapeDtypeStruct((_NP, _D), jnp.float32),
    ),
    mesh=_mesh,
    scratch_types=[
        pltpu.VMEM((_SUBE,), jnp.int32),
        pltpu.VMEM((_SUBE,), jnp.int32),
        pltpu.VMEM((_SUBE,), jnp.int32),
        pltpu.VMEM((_SUBE,), jnp.int32),
        pltpu.VMEM((_SUBE,), jnp.int32),
        pltpu.VMEM((_SUBE,), jnp.int32),
        pltpu.VMEM((_SUBE,), jnp.float32),
        pltpu.VMEM((_SUBE,), jnp.float32),
        pltpu.VMEM((_SUBE,), jnp.float32),
        pltpu.VMEM((_SUBE,), jnp.float32),
        pltpu.VMEM((_XPW,), jnp.int32),
        pltpu.VMEM((128, _D), jnp.float32),
        pltpu.VMEM_SHARED((_CAP,), jnp.float32),
        pltpu.VMEM_SHARED((_CAP,), jnp.float32),
        pltpu.SemaphoreType.DMA,
        pltpu.SemaphoreType.DMA,
        pltpu.SemaphoreType.DMA,
    ],
)
def _sc_graph(esrc_hbm, edst_hbm, etyp_hbm, xids_hbm, emb_hbm,
              a_out, x_out,
              esrc, edst, etyp, segd, sega0, sega1, val0, val1, cnt0, cnt1,
              xidx, xrows, counts_sh, at_sh, sem_ld, sem_sc, sem_g):
    cid = lax.axis_index("c")
    tid = lax.axis_index("s")
    wid = tid * 2 + cid
    segas = (sega0, sega1)
    vals = (val0, val1)
    cnts = (cnt0, cnt1)

    def _zero_i(i, carry):
        val0[pl.ds(i * 16, 16)] = jnp.zeros((16,), jnp.float32)
        return carry
    lax.fori_loop(0, _SUBE // 16, _zero_i, None)
    for shared in (counts_sh, at_sh):
        for k in range(6):
            pltpu.sync_copy(val0, shared.at[pl.ds(tid * _SLAB + k * _SUBE, _SUBE)])
        pltpu.sync_copy(
            val0.at[pl.ds(0, _SLAB - 6 * _SUBE)],
            shared.at[pl.ds(tid * _SLAB + 6 * _SUBE, _SLAB - 6 * _SUBE)])

    @pl.when(wid < _XW)
    def _():
        pltpu.sync_copy(xids_hbm.at[pl.ds(wid * _XPW, _XPW)], xidx)
        for off, n in ((0, 128), (128, 128), (256, 128), (384, 16)):
            pltpu.async_copy(emb_hbm.at[xidx.at[pl.ds(off, n)]],
                             xrows.at[pl.ds(0, n)], sem_ld).wait()
            pltpu.sync_copy(xrows.at[pl.ds(0, n)],
                            x_out.at[pl.ds(wid * _XPW + off, n)])

    @pl.when(wid == _XW)
    def _():
        pltpu.sync_copy(xids_hbm.at[pl.ds(0, 128)], xidx.at[pl.ds(0, 128)])
        pltpu.async_copy(emb_hbm.at[xidx.at[pl.ds(0, 128)]],
                         xrows, sem_ld).wait()
        pltpu.sync_copy(xrows, x_out.at[pl.ds(_N, 128)])

    @pl.when(wid == _XW + 1)
    def _():
        pltpu.sync_copy(xids_hbm.at[pl.ds(128, 112)], xidx.at[pl.ds(0, 112)])
        pltpu.async_copy(emb_hbm.at[xidx.at[pl.ds(0, 112)]],
                         xrows.at[pl.ds(0, 112)], sem_ld).wait()
        pltpu.sync_copy(xrows.at[pl.ds(0, 112)],
                        x_out.at[pl.ds(_N + 128, 112)])

    def _ones_i(i, carry):
        val0[pl.ds(i * 16, 16)] = jnp.ones((16,), jnp.float32)
        val1[pl.ds(i * 16, 16)] = jnp.ones((16,), jnp.float32)
        return carry
    lax.fori_loop(0, _SUBE // 16, _ones_i, None)

    plsc.subcore_barrier()

    def _loads_a(k):
        eb = tid * _EPT + k * _SUBE
        la = pltpu.async_copy(edst_hbm.at[pl.ds(eb, _SUBE)], edst, sem_ld)
        lb = pltpu.async_copy(etyp_hbm.at[pl.ds(eb, _SUBE)], etyp, sem_ld)
        return la, lb

    def _seg_a(k):
        sega = segas[k % 2]

        def _seg_i(i, carry):
            sl = pl.ds(i * 16, 16)
            sega[sl] = etyp[sl] * _NP + edst[sl]
            return carry
        lax.fori_loop(0, _SUBE // 16, _seg_i, None)
        return sega

    pend_ld = _loads_a(0)
    pend_sc = None
    for k in range(_NSUB):
        for c in pend_ld:
            c.wait()
        sega = _seg_a(k)
        if k + 1 < _NSUB:
            pend_ld = _loads_a(k + 1)
        if pend_sc is not None:
            pend_sc.wait()
        pend_sc = pltpu.async_copy(vals[k % 2], counts_sh.at[sega], sem_sc,
                                   add=True)
    pend_sc.wait()

    plsc.subcore_barrier()

    def _loads_b(k):
        eb = tid * _EPT + k * _SUBE
        la = pltpu.async_copy(esrc_hbm.at[pl.ds(eb, _SUBE)], esrc, sem_ld)
        lb = pltpu.async_copy(edst_hbm.at[pl.ds(eb, _SUBE)], edst, sem_ld)
        lc = pltpu.async_copy(etyp_hbm.at[pl.ds(eb, _SUBE)], etyp, sem_ld)
        return la, lb, lc

    def _seg_b(k):
        sega = segas[k % 2]

        def _seg_i(i, carry):
            sl = pl.ds(i * 16, 16)
            t = etyp[sl] * _NP
            segd[sl] = t + edst[sl]
            sega[sl] = t + esrc[sl]
            return carry
        lax.fori_loop(0, _SUBE // 16, _seg_i, None)
        return sega

    def _recip(k):
        cnt, val = cnts[k % 2], vals[k % 2]

        def _w_i(i, carry):
            sl = pl.ds(i * 16, 16)
            val[sl] = 1.0 / cnt[sl]
            return carry
        lax.fori_loop(0, _SUBE // 16, _w_i, None)
        return val

    pend_ld = _loads_b(0)
    for c in pend_ld:
        c.wait()
    sega = _seg_b(0)
    pend_g = pltpu.async_copy(counts_sh.at[segd], cnts[0], sem_g)
    pend_sc = None
    for k in range(_NSUB):
        if k + 1 < _NSUB:
            for c in _loads_b(k + 1):
                c.wait()
        pend_g.wait()
        if k + 1 < _NSUB:
            next_sega = _seg_b(k + 1)
            pend_g = pltpu.async_copy(counts_sh.at[segd], cnts[(k + 1) % 2],
                                      sem_g)
        val = _recip(k)
        if pend_sc is not None:
            pend_sc.wait()
        pend_sc = pltpu.async_copy(val, at_sh.at[sega], sem_sc, add=True)
        if k + 1 < _NSUB:
            sega = next_sega
    pend_sc.wait()

    plsc.subcore_barrier()

    @pl.when(cid == 0)
    def _():
        chunks = [(k * _SUBE, _SUBE) for k in range(6)]
        chunks.append((6 * _SUBE, _SLAB - 6 * _SUBE))
        pend = None
        for i, (off, n) in enumerate(chunks):
            buf = cnts[i % 2]
            pltpu.sync_copy(at_sh.at[pl.ds(tid * _SLAB + off, n)],
                            buf.at[pl.ds(0, n)])
            if pend is not None:
                pend.wait()
            pend = pltpu.async_copy(buf.at[pl.ds(0, n)],
                                    a_out.at[pl.ds(tid * _SLAB + off, n)],
                                    sem_sc)
        pend.wait()


def _tc_body(x_ref, at_ref, compt_ref, bases_ref, root_ref, bias_ref, o_ref):
    x = x_ref[...]
    s = jnp.dot(at_ref[...], x,
                preferred_element_type=jnp.float32)
    t = jnp.dot(compt_ref[...], s,
                preferred_element_type=jnp.float32)
    agg = jnp.zeros((1, _D), jnp.float32)
    for b in range(_B):
        agg = agg + jnp.dot(t[b:b + 1, :], bases_ref[b],
                            preferred_element_type=jnp.float32)
    xs = jnp.sum(x[:_N], axis=0, keepdims=True)
    g = agg + jnp.dot(xs, root_ref[...],
                      preferred_element_type=jnp.float32)
    g = g + float(_N) * bias_ref[...]
    nrm = jnp.sqrt(jnp.sum(g * g))
    o_ref[...] = g / jnp.maximum(nrm, 1e-5)


_tc_readout = pl.pallas_call(
    _tc_body,
    out_shape=jax.ShapeDtypeStruct((1, _D), jnp.float32),
)


def kernel(node_final_id, edge_index, edge_type, embedding, comp, bases, root, bias):
    edge_index = edge_index.astype(jnp.int32)
    a_flat, x_pad = _sc_graph(edge_index[0], edge_index[1],
                              edge_type.astype(jnp.int32),
                              node_final_id.astype(jnp.int32),
                              embedding)
    a_t = a_flat.reshape(_R, _NP)
    return _tc_readout(x_pad, a_t, comp.T, bases, root, bias.reshape(1, _D))


# speedup vs baseline: 32.7247x; 1.0041x over previous
import functools

import jax
import jax.numpy as jnp
from jax import lax
from jax.experimental import pallas as pl
from jax.experimental.pallas import tpu as pltpu
from jax.experimental.pallas import tpu_sc as plsc

_N = 10000
_NP = 10240
_E = 320000
_D = 128
_R = 39
_B = 8

_NT = 16
_SUBE = 4992
_NSUB = 4
_EPT = _SUBE * _NSUB
_EMAIN = _EPT * _NT
_NMINI = 4
_CAP = _R * _NP
_SLAB = _CAP // _NT

_XW = 25
_XPW = _N // _XW

_mesh = plsc.VectorSubcoreMesh(
    core_axis_name="c", subcore_axis_name="s", num_cores=2, num_subcores=_NT
)


@functools.partial(
    pl.kernel,
    out_type=(
        jax.ShapeDtypeStruct((_CAP,), jnp.float32),
        jax.ShapeDtypeStruct((_NP, _D), jnp.float32),
    ),
    mesh=_mesh,
    scratch_types=[
        pltpu.VMEM((2, _SUBE), jnp.int32),
        pltpu.VMEM((_SUBE,), jnp.int32),
        pltpu.VMEM((_SUBE,), jnp.int32),
        pltpu.VMEM((_SUBE,), jnp.int32),
        pltpu.VMEM((_SUBE,), jnp.int32),
        pltpu.VMEM((_SUBE,), jnp.float32),
        pltpu.VMEM((_SUBE,), jnp.float32),
        pltpu.VMEM((_SUBE,), jnp.float32),
        pltpu.VMEM((_SUBE,), jnp.float32),
        pltpu.VMEM((2, 128), jnp.int32),
        pltpu.VMEM((128,), jnp.int32),
        pltpu.VMEM((128,), jnp.int32),
        pltpu.VMEM((128,), jnp.int32),
        pltpu.VMEM((128,), jnp.float32),
        pltpu.VMEM((128,), jnp.float32),
        pltpu.VMEM((_XPW,), jnp.int32),
        pltpu.VMEM((64, _D), jnp.float32),
        pltpu.VMEM((64, _D), jnp.float32),
        pltpu.VMEM_SHARED((_CAP,), jnp.float32),
        pltpu.VMEM_SHARED((_CAP,), jnp.float32),
        pltpu.SemaphoreType.DMA,
        pltpu.SemaphoreType.DMA,
        pltpu.SemaphoreType.DMA,
    ],
)
def _sc_graph(edge_hbm, etyp_hbm, xids_hbm, emb_hbm,
              a_out, x_out,
              ebuf, etyp, segd, sega0, sega1, val0, val1, cnt0, cnt1,
              ebmini, etmini, smini, dmini, vmini, cmini,
              xidx, xrows0, xrows1, counts_sh, at_sh, sem_ld, sem_sc, sem_g):
    cid = lax.axis_index("c")
    tid = lax.axis_index("s")
    wid = tid * 2 + cid
    segas = (sega0, sega1)
    vals = (val0, val1)
    cnts = (cnt0, cnt1)
    xrows = (xrows0, xrows1)

    def _zero_i(i, carry):
        val0[pl.ds(i * 16, 16)] = jnp.zeros((16,), jnp.float32)
        return carry
    lax.fori_loop(0, _SUBE // 16, _zero_i, None)
    zcopies = []
    for shared in (counts_sh, at_sh):
        for k in range(5):
            zcopies.append(pltpu.async_copy(
                val0, shared.at[pl.ds(tid * _SLAB + k * _SUBE, _SUBE)],
                sem_sc))

    def _xgather(id_off, row_off, sizes):
        chunks = []
        o = 0
        for n in sizes:
            chunks.append((o, n))
            o += n
        pend = pltpu.async_copy(
            emb_hbm.at[xidx.at[pl.ds(id_off + chunks[0][0], chunks[0][1])]],
            xrows[0].at[pl.ds(0, chunks[0][1])], sem_ld)
        for j, (off, n) in enumerate(chunks):
            pend.wait()
            if j + 1 < len(chunks):
                noff, nn = chunks[j + 1]
                pend = pltpu.async_copy(
                    emb_hbm.at[xidx.at[pl.ds(id_off + noff, nn)]],
                    xrows[(j + 1) % 2].at[pl.ds(0, nn)], sem_ld)
            pltpu.sync_copy(xrows[j % 2].at[pl.ds(0, n)],
                            x_out.at[pl.ds(row_off + off, n)])

    @pl.when(wid < _XW)
    def _():
        pltpu.sync_copy(xids_hbm.at[pl.ds(wid * _XPW, _XPW)], xidx)
        _xgather(0, wid * _XPW, [64] * 6 + [16])

    @pl.when(wid == _XW)
    def _():
        pltpu.sync_copy(xids_hbm.at[pl.ds(0, 128)], xidx.at[pl.ds(0, 128)])
        _xgather(0, _N, [64, 64])

    @pl.when(wid == _XW + 1)
    def _():
        pltpu.sync_copy(xids_hbm.at[pl.ds(128, 112)], xidx.at[pl.ds(0, 112)])
        _xgather(0, _N + 128, [64, 48])

    for c in zcopies:
        c.wait()

    def _ones_i(i, carry):
        val0[pl.ds(i * 16, 16)] = jnp.ones((16,), jnp.float32)
        val1[pl.ds(i * 16, 16)] = jnp.ones((16,), jnp.float32)
        return carry
    lax.fori_loop(0, _SUBE // 16, _ones_i, None)

    def _ones_m(i, carry):
        vmini[pl.ds(i * 16, 16)] = jnp.ones((16,), jnp.float32)
        return carry
    lax.fori_loop(0, 8, _ones_m, None)

    plsc.subcore_barrier()

    def _loads(k):
        eb = tid * _EPT + k * _SUBE
        la = pltpu.async_copy(edge_hbm.at[:, pl.ds(eb, _SUBE)], ebuf, sem_ld)
        lb = pltpu.async_copy(etyp_hbm.at[pl.ds(eb, _SUBE)], etyp, sem_ld)
        return la, lb

    def _seg_a(k):
        sega = segas[k % 2]

        def _seg_i(i, carry):
            sl = pl.ds(i * 16, 16)
            sega[sl] = etyp[sl] * _NP + ebuf[1, sl]
            return carry
        lax.fori_loop(0, _SUBE // 16, _seg_i, None)
        return sega

    pend_ld = _loads(0)
    pend_sc = None
    for k in range(_NSUB):
        for c in pend_ld:
            c.wait()
        sega = _seg_a(k)
        if k + 1 < _NSUB:
            pend_ld = _loads(k + 1)
        if pend_sc is not None:
            pend_sc.wait()
        pend_sc = pltpu.async_copy(vals[k % 2], counts_sh.at[sega], sem_sc,
                                   add=True)
    pend_sc.wait()

    @pl.when(tid < _NMINI)
    def _():
        ebm = _EMAIN + tid * 128
        pltpu.sync_copy(edge_hbm.at[:, pl.ds(ebm, 128)], ebmini)
        pltpu.sync_copy(etyp_hbm.at[pl.ds(ebm, 128)], etmini)

        def _mi(i, carry):
            sl = pl.ds(i * 16, 16)
            smini[sl] = etmini[sl] * _NP + ebmini[1, sl]
            return carry
        lax.fori_loop(0, 8, _mi, None)
        pltpu.sync_copy(vmini, counts_sh.at[smini], add=True)

    plsc.subcore_barrier()

    def _seg_b(k):
        sega = segas[k % 2]

        def _seg_i(i, carry):
            sl = pl.ds(i * 16, 16)
            t = etyp[sl] * _NP
            segd[sl] = t + ebuf[1, sl]
            sega[sl] = t + ebuf[0, sl]
            return carry
        lax.fori_loop(0, _SUBE // 16, _seg_i, None)
        return sega

    def _recip(k):
        cnt, val = cnts[k % 2], vals[k % 2]

        def _w_i(i, carry):
            sl = pl.ds(i * 16, 16)
            val[sl] = 1.0 / cnt[sl]
            return carry
        lax.fori_loop(0, _SUBE // 16, _w_i, None)
        return val

    pend_ld = _loads(0)
    for c in pend_ld:
        c.wait()
    sega = _seg_b(0)
    pend_g = pltpu.async_copy(counts_sh.at[segd], cnts[0], sem_g)
    pend_sc = None
    for k in range(_NSUB):
        if k + 1 < _NSUB:
            for c in _loads(k + 1):
                c.wait()
        pend_g.wait()
        if k + 1 < _NSUB:
            next_sega = _seg_b(k + 1)
            pend_g = pltpu.async_copy(counts_sh.at[segd], cnts[(k + 1) % 2],
                                      sem_g)
        val = _recip(k)
        if pend_sc is not None:
            pend_sc.wait()
        pend_sc = pltpu.async_copy(val, at_sh.at[sega], sem_sc, add=True)
        if k + 1 < _NSUB:
            sega = next_sega
    pend_sc.wait()

    @pl.when(tid < _NMINI)
    def _():
        ebm = _EMAIN + tid * 128
        pltpu.sync_copy(edge_hbm.at[:, pl.ds(ebm, 128)], ebmini)
        pltpu.sync_copy(etyp_hbm.at[pl.ds(ebm, 128)], etmini)

        def _mi(i, carry):
            sl = pl.ds(i * 16, 16)
            t = etmini[sl] * _NP
            dmini[sl] = t + ebmini[1, sl]
            smini[sl] = t + ebmini[0, sl]
            return carry
        lax.fori_loop(0, 8, _mi, None)
        pltpu.async_copy(counts_sh.at[dmini], cmini, sem_g).wait()

        def _wm(i, carry):
            sl = pl.ds(i * 16, 16)
            vmini[sl] = 1.0 / cmini[sl]
            return carry
        lax.fori_loop(0, 8, _wm, None)
        pltpu.sync_copy(vmini, at_sh.at[smini], add=True)

    plsc.subcore_barrier()

    @pl.when(cid == 0)
    def _():
        pend = None
        for i in range(5):
            off = tid * _SLAB + i * _SUBE
            buf = cnts[i % 2]
            pltpu.sync_copy(at_sh.at[pl.ds(off, _SUBE)], buf)
            if pend is not None:
                pend.wait()
            pend = pltpu.async_copy(buf, a_out.at[pl.ds(off, _SUBE)], sem_sc)
        pend.wait()


def _tc_body(x_ref, at_ref, compt_ref, bases_ref, root_ref, bias_ref, o_ref):
    x = x_ref[...]
    s = jnp.dot(at_ref[...], x,
                preferred_element_type=jnp.float32)
    t = jnp.dot(compt_ref[...], s,
                preferred_element_type=jnp.float32)
    agg = jnp.zeros((1, _D), jnp.float32)
    for b in range(_B):
        agg = agg + jnp.dot(t[b:b + 1, :], bases_ref[b],
                            preferred_element_type=jnp.float32)
    xs = jnp.sum(x[:_N], axis=0, keepdims=True)
    g = agg + jnp.dot(xs, root_ref[...],
                      preferred_element_type=jnp.float32)
    g = g + float(_N) * bias_ref[...]
    nrm = jnp.sqrt(jnp.sum(g * g))
    o_ref[...] = g / jnp.maximum(nrm, 1e-5)


_tc_readout = pl.pallas_call(
    _tc_body,
    out_shape=jax.ShapeDtypeStruct((1, _D), jnp.float32),
)


def kernel(node_final_id, edge_index, edge_type, embedding, comp, bases, root, bias):
    a_flat, x_pad = _sc_graph(edge_index.astype(jnp.int32),
                              edge_type.astype(jnp.int32),
                              node_final_id.astype(jnp.int32),
                              embedding)
    a_t = a_flat.reshape(_R, _NP)
    return _tc_readout(x_pad, a_t, comp.T, bases, root, bias.reshape(1, _D))


# speedup vs baseline: 34.8721x; 1.0656x over previous
import functools

import jax
import jax.numpy as jnp
from jax import lax
from jax.experimental import pallas as pl
from jax.experimental.pallas import tpu as pltpu
from jax.experimental.pallas import tpu_sc as plsc

_N = 10000
_NP = 10240
_E = 320000
_D = 128
_R = 39
_B = 8

_NT = 16
_SUBE = 4992
_NSUB = 4
_EPT = _SUBE * _NSUB
_EMAIN = _EPT * _NT
_NMINI = 4
_CAP = _R * _NP
_SLAB = _CAP // _NT

_XPW = 320

_mesh = plsc.VectorSubcoreMesh(
    core_axis_name="c", subcore_axis_name="s", num_cores=2, num_subcores=_NT
)


@functools.partial(
    pl.kernel,
    out_type=(
        jax.ShapeDtypeStruct((_CAP,), jnp.float32),
        jax.ShapeDtypeStruct((_NP, _D), jnp.float32),
    ),
    mesh=_mesh,
    scratch_types=[
        pltpu.VMEM((2, _SUBE), jnp.int32),
        pltpu.VMEM((_SUBE,), jnp.int32),
        pltpu.VMEM((_SUBE,), jnp.int32),
        pltpu.VMEM((_SUBE,), jnp.int32),
        pltpu.VMEM((_SUBE,), jnp.int32),
        pltpu.VMEM((_SUBE,), jnp.float32),
        pltpu.VMEM((_SUBE,), jnp.float32),
        pltpu.VMEM((_SUBE,), jnp.float32),
        pltpu.VMEM((_SUBE,), jnp.float32),
        pltpu.VMEM((2, 128), jnp.int32),
        pltpu.VMEM((128,), jnp.int32),
        pltpu.VMEM((128,), jnp.int32),
        pltpu.VMEM((128,), jnp.int32),
        pltpu.VMEM((128,), jnp.float32),
        pltpu.VMEM((128,), jnp.float32),
        pltpu.VMEM((320,), jnp.int32),
        pltpu.VMEM((64, _D), jnp.float32),
        pltpu.VMEM((64, _D), jnp.float32),
        pltpu.VMEM_SHARED((_CAP,), jnp.float32),
        pltpu.VMEM_SHARED((_CAP,), jnp.float32),
        pltpu.SemaphoreType.DMA,
        pltpu.SemaphoreType.DMA,
        pltpu.SemaphoreType.DMA,
        pltpu.SemaphoreType.DMA,
    ],
)
def _sc_graph(edge_hbm, etyp_hbm, xids_hbm, emb_hbm,
              a_out, x_out,
              ebuf, etyp, segd, sega0, sega1, val0, val1, cnt0, cnt1,
              ebmini, etmini, smini, dmini, vmini, cmini,
              xidx, xrows0, xrows1, counts_sh, at_sh, sem_ld, sem_sc, sem_g,
              sem_xg):
    cid = lax.axis_index("c")
    tid = lax.axis_index("s")
    wid = tid * 2 + cid
    segas = (sega0, sega1)
    vals = (val0, val1)
    cnts = (cnt0, cnt1)
    xrows = (xrows0, xrows1)

    scope = jax.named_scope
    def _zero_i(i, carry):
        val0[pl.ds(i * 16, 16)] = jnp.zeros((16,), jnp.float32)
        return carry
    lax.fori_loop(0, _SUBE // 16, _zero_i, None)
    zcopies = []
    for shared in (counts_sh, at_sh):
        for k in range(5):
            zcopies.append(pltpu.async_copy(
                val0, shared.at[pl.ds(tid * _SLAB + k * _SUBE, _SUBE)],
                sem_sc))

    @pl.when(wid < 31)
    def _():
        pltpu.sync_copy(xids_hbm.at[pl.ds(wid * _XPW, _XPW)], xidx)

    @pl.when(wid == 31)
    def _():
        pltpu.sync_copy(xids_hbm.at[pl.ds(9920, 80)], xidx.at[pl.ds(0, 80)])
        pltpu.sync_copy(xids_hbm.at[pl.ds(0, 240)], xidx.at[pl.ds(80, 240)])

    def _xg_fire(j):
        return pltpu.async_copy(emb_hbm.at[xidx.at[pl.ds(j * 64, 64)]],
                                xrows[j % 2], sem_xg)

    def _xg_write(j):
        pltpu.sync_copy(xrows[j % 2], x_out.at[pl.ds(wid * _XPW + j * 64, 64)])

    pend_xg = _xg_fire(0)

    for c in zcopies:
        c.wait()

    def _ones_i(i, carry):
        val0[pl.ds(i * 16, 16)] = jnp.ones((16,), jnp.float32)
        val1[pl.ds(i * 16, 16)] = jnp.ones((16,), jnp.float32)
        return carry
    lax.fori_loop(0, _SUBE // 16, _ones_i, None)

    def _ones_m(i, carry):
        vmini[pl.ds(i * 16, 16)] = jnp.ones((16,), jnp.float32)
        return carry
    lax.fori_loop(0, 8, _ones_m, None)

    plsc.subcore_barrier()

    _sA = scope("phaseA"); _sA.__enter__()
    def _loads(k):
        eb = tid * _EPT + k * _SUBE
        la = pltpu.async_copy(edge_hbm.at[:, pl.ds(eb, _SUBE)], ebuf, sem_ld)
        lb = pltpu.async_copy(etyp_hbm.at[pl.ds(eb, _SUBE)], etyp, sem_ld)
        return la, lb

    def _seg_a(k):
        sega = segas[k % 2]

        def _seg_i(i, carry):
            sl = pl.ds(i * 16, 16)
            sega[sl] = etyp[sl] * _NP + ebuf[1, sl]
            return carry
        lax.fori_loop(0, _SUBE // 16, _seg_i, None)
        return sega

    pend_ld = _loads(0)
    pend_sc = None
    for k in range(_NSUB):
        for c in pend_ld:
            c.wait()
        sega = _seg_a(k)
        if k + 1 < _NSUB:
            pend_ld = _loads(k + 1)
        if pend_sc is not None:
            pend_sc.wait()
        pend_sc = pltpu.async_copy(vals[k % 2], counts_sh.at[sega], sem_sc,
                                   add=True)
        pend_xg.wait()
        nxt = _xg_fire(k + 1)
        _xg_write(k)
        pend_xg = nxt
    pend_sc.wait()
    pend_xg.wait()
    _xg_write(4)

    @pl.when(tid < _NMINI)
    def _():
        ebm = _EMAIN + tid * 128
        pltpu.sync_copy(edge_hbm.at[:, pl.ds(ebm, 128)], ebmini)
        pltpu.sync_copy(etyp_hbm.at[pl.ds(ebm, 128)], etmini)

        def _mi(i, carry):
            sl = pl.ds(i * 16, 16)
            smini[sl] = etmini[sl] * _NP + ebmini[1, sl]
            return carry
        lax.fori_loop(0, 8, _mi, None)
        pltpu.sync_copy(vmini, counts_sh.at[smini], add=True)

    _sA.__exit__(None, None, None)
    plsc.subcore_barrier()

    _sB = scope("phaseB"); _sB.__enter__()
    def _seg_b(k):
        sega = segas[k % 2]

        def _seg_i(i, carry):
            sl = pl.ds(i * 16, 16)
            t = etyp[sl] * _NP
            segd[sl] = t + ebuf[1, sl]
            sega[sl] = t + ebuf[0, sl]
            return carry
        lax.fori_loop(0, _SUBE // 16, _seg_i, None)
        return sega

    def _recip(k):
        cnt, val = cnts[k % 2], vals[k % 2]

        def _w_i(i, carry):
            sl = pl.ds(i * 16, 16)
            val[sl] = 1.0 / cnt[sl]
            return carry
        lax.fori_loop(0, _SUBE // 16, _w_i, None)
        return val

    pend_ld = _loads(0)
    for c in pend_ld:
        c.wait()
    sega = _seg_b(0)
    pend_g = pltpu.async_copy(counts_sh.at[segd], cnts[0], sem_g)
    pend_sc = None
    for k in range(_NSUB):
        if k + 1 < _NSUB:
            for c in _loads(k + 1):
                c.wait()
        pend_g.wait()
        if k + 1 < _NSUB:
            next_sega = _seg_b(k + 1)
            pend_g = pltpu.async_copy(counts_sh.at[segd], cnts[(k + 1) % 2],
                                      sem_g)
        val = _recip(k)
        if pend_sc is not None:
            pend_sc.wait()
        pend_sc = pltpu.async_copy(val, at_sh.at[sega], sem_sc, add=True)
        if k + 1 < _NSUB:
            sega = next_sega
    pend_sc.wait()

    @pl.when(tid < _NMINI)
    def _():
        ebm = _EMAIN + tid * 128
        pltpu.sync_copy(edge_hbm.at[:, pl.ds(ebm, 128)], ebmini)
        pltpu.sync_copy(etyp_hbm.at[pl.ds(ebm, 128)], etmini)

        def _mi(i, carry):
            sl = pl.ds(i * 16, 16)
            t = etmini[sl] * _NP
            dmini[sl] = t + ebmini[1, sl]
            smini[sl] = t + ebmini[0, sl]
            return carry
        lax.fori_loop(0, 8, _mi, None)
        pltpu.async_copy(counts_sh.at[dmini], cmini, sem_g).wait()

        def _wm(i, carry):
            sl = pl.ds(i * 16, 16)
            vmini[sl] = 1.0 / cmini[sl]
            return carry
        lax.fori_loop(0, 8, _wm, None)
        pltpu.sync_copy(vmini, at_sh.at[smini], add=True)

    _sB.__exit__(None, None, None)
    plsc.subcore_barrier()

    @pl.when(cid == 0)
    def _():
        pend = None
        for i in range(5):
            off = tid * _SLAB + i * _SUBE
            buf = cnts[i % 2]
            pltpu.sync_copy(at_sh.at[pl.ds(off, _SUBE)], buf)
            if pend is not None:
                pend.wait()
            pend = pltpu.async_copy(buf, a_out.at[pl.ds(off, _SUBE)], sem_sc)
        pend.wait()


def _tc_body(x_ref, at_ref, compt_ref, bases_ref, root_ref, bias_ref, o_ref):
    x = x_ref[...]
    s = jnp.dot(at_ref[...], x,
                preferred_element_type=jnp.float32)
    t = jnp.dot(compt_ref[...], s,
                preferred_element_type=jnp.float32)
    agg = jnp.zeros((1, _D), jnp.float32)
    for b in range(_B):
        agg = agg + jnp.dot(t[b:b + 1, :], bases_ref[b],
                            preferred_element_type=jnp.float32)
    xs = jnp.sum(x[:_N], axis=0, keepdims=True)
    g = agg + jnp.dot(xs, root_ref[...],
                      preferred_element_type=jnp.float32)
    g = g + float(_N) * bias_ref[...]
    nrm = jnp.sqrt(jnp.sum(g * g))
    o_ref[...] = g / jnp.maximum(nrm, 1e-5)


_tc_readout = pl.pallas_call(
    _tc_body,
    out_shape=jax.ShapeDtypeStruct((1, _D), jnp.float32),
)


def kernel(node_final_id, edge_index, edge_type, embedding, comp, bases, root, bias):
    a_flat, x_pad = _sc_graph(edge_index.astype(jnp.int32),
                              edge_type.astype(jnp.int32),
                              node_final_id.astype(jnp.int32),
                              embedding)
    a_t = a_flat.reshape(_R, _NP)
    return _tc_readout(x_pad, a_t, comp.T, bases, root, bias.reshape(1, _D))


# speedup vs baseline: 40.8806x; 1.1723x over previous
import functools

import jax
import jax.numpy as jnp
from jax import lax
from jax.experimental import pallas as pl
from jax.experimental.pallas import tpu as pltpu
from jax.experimental.pallas import tpu_sc as plsc

_N = 10000
_NP = 10240
_E = 320000
_D = 128
_R = 39
_B = 8

_NT = 16
_SUBE = 4992
_NSUB = 4
_EPT = _SUBE * _NSUB
_EMAIN = _EPT * _NT
_NMINI = 4
_CAP = _R * _NP
_SLAB = _CAP // _NT

_XPW = 320

_mesh = plsc.VectorSubcoreMesh(
    core_axis_name="c", subcore_axis_name="s", num_cores=2, num_subcores=_NT
)


@functools.partial(
    pl.kernel,
    out_type=(
        jax.ShapeDtypeStruct((_CAP,), jnp.float32),
        jax.ShapeDtypeStruct((_CAP,), jnp.float32),
        jax.ShapeDtypeStruct((_NP, _D), jnp.float32),
    ),
    mesh=_mesh,
    scratch_types=[
        pltpu.VMEM((2, _SUBE), jnp.int32),
        pltpu.VMEM((_SUBE,), jnp.int32),
        pltpu.VMEM((_SUBE,), jnp.int32),
        pltpu.VMEM((_SUBE,), jnp.int32),
        pltpu.VMEM((_SUBE,), jnp.int32),
        pltpu.VMEM((_SUBE,), jnp.float32),
        pltpu.VMEM((_SUBE,), jnp.float32),
        pltpu.VMEM((_SUBE,), jnp.float32),
        pltpu.VMEM((_SUBE,), jnp.float32),
        pltpu.VMEM((2, 128), jnp.int32),
        pltpu.VMEM((128,), jnp.int32),
        pltpu.VMEM((128,), jnp.int32),
        pltpu.VMEM((128,), jnp.int32),
        pltpu.VMEM((128,), jnp.float32),
        pltpu.VMEM((128,), jnp.float32),
        pltpu.VMEM((320,), jnp.int32),
        pltpu.VMEM((64, _D), jnp.float32),
        pltpu.VMEM((64, _D), jnp.float32),
        pltpu.VMEM_SHARED((_CAP,), jnp.float32),
        pltpu.VMEM_SHARED((_CAP,), jnp.float32),
        pltpu.SemaphoreType.DMA,
        pltpu.SemaphoreType.DMA,
        pltpu.SemaphoreType.DMA,
        pltpu.SemaphoreType.DMA,
    ],
)
def _sc_graph(edge_hbm, etyp_hbm, xids_hbm, emb_hbm,
              a_out0, a_out1, x_out,
              ebuf, etyp, segd, sega0, sega1, val0, val1, cnt0, cnt1,
              ebmini, etmini, smini, dmini, vmini, cmini,
              xidx, xrows0, xrows1, counts_sh, at_sh, sem_ld, sem_sc, sem_g,
              sem_xg):
    cid = lax.axis_index("c")
    tid = lax.axis_index("s")
    wid = tid * 2 + cid
    segas = (sega0, sega1)
    vals = (val0, val1)
    cnts = (cnt0, cnt1)
    xrows = (xrows0, xrows1)

    scope = jax.named_scope
    def _zero_i(i, carry):
        val0[pl.ds(i * 16, 16)] = jnp.zeros((16,), jnp.float32)
        return carry
    lax.fori_loop(0, _SUBE // 16, _zero_i, None)
    zcopies = []
    for shared in (counts_sh, at_sh):
        for k in range(5):
            zcopies.append(pltpu.async_copy(
                val0, shared.at[pl.ds(tid * _SLAB + k * _SUBE, _SUBE)],
                sem_sc))

    @pl.when(wid < 31)
    def _():
        pltpu.sync_copy(xids_hbm.at[pl.ds(wid * _XPW, _XPW)], xidx)

    @pl.when(wid == 31)
    def _():
        pltpu.sync_copy(xids_hbm.at[pl.ds(9920, 80)], xidx.at[pl.ds(0, 80)])
        pltpu.sync_copy(xids_hbm.at[pl.ds(0, 240)], xidx.at[pl.ds(80, 240)])

    def _xg_fire(j):
        return pltpu.async_copy(emb_hbm.at[xidx.at[pl.ds(j * 64, 64)]],
                                xrows[j % 2], sem_xg)

    def _xg_write(j):
        pltpu.sync_copy(xrows[j % 2], x_out.at[pl.ds(wid * _XPW + j * 64, 64)])

    pend_xg = _xg_fire(0)

    for c in zcopies:
        c.wait()

    def _ones_i(i, carry):
        val0[pl.ds(i * 16, 16)] = jnp.ones((16,), jnp.float32)
        val1[pl.ds(i * 16, 16)] = jnp.ones((16,), jnp.float32)
        return carry
    lax.fori_loop(0, _SUBE // 16, _ones_i, None)

    def _ones_m(i, carry):
        vmini[pl.ds(i * 16, 16)] = jnp.ones((16,), jnp.float32)
        return carry
    lax.fori_loop(0, 8, _ones_m, None)

    plsc.subcore_barrier()

    _sA = scope("phaseA"); _sA.__enter__()
    def _loads(k):
        eb = tid * _EPT + k * _SUBE
        la = pltpu.async_copy(edge_hbm.at[:, pl.ds(eb, _SUBE)], ebuf, sem_ld)
        lb = pltpu.async_copy(etyp_hbm.at[pl.ds(eb, _SUBE)], etyp, sem_ld)
        return la, lb

    def _seg_a(k):
        sega = segas[k % 2]

        def _seg_i(i, carry):
            sl = pl.ds(i * 16, 16)
            sega[sl] = etyp[sl] * _NP + ebuf[1, sl]
            return carry
        lax.fori_loop(0, _SUBE // 16, _seg_i, None)
        return sega

    pend_ld = _loads(0)
    pend_sc = None
    for k in range(_NSUB):
        for c in pend_ld:
            c.wait()
        sega = _seg_a(k)
        if k + 1 < _NSUB:
            pend_ld = _loads(k + 1)
        if pend_sc is not None:
            pend_sc.wait()
        pend_sc = pltpu.async_copy(vals[k % 2], counts_sh.at[sega], sem_sc,
                                   add=True)
        pend_xg.wait()
        nxt = _xg_fire(k + 1)
        _xg_write(k)
        pend_xg = nxt
    pend_sc.wait()
    pend_xg.wait()
    _xg_write(4)

    @pl.when(tid < _NMINI)
    def _():
        ebm = _EMAIN + tid * 128
        pltpu.sync_copy(edge_hbm.at[:, pl.ds(ebm, 128)], ebmini)
        pltpu.sync_copy(etyp_hbm.at[pl.ds(ebm, 128)], etmini)

        def _mi(i, carry):
            sl = pl.ds(i * 16, 16)
            smini[sl] = etmini[sl] * _NP + ebmini[1, sl]
            return carry
        lax.fori_loop(0, 8, _mi, None)
        pltpu.sync_copy(vmini, counts_sh.at[smini], add=True)

    _sA.__exit__(None, None, None)
    plsc.subcore_barrier()

    _sB = scope("phaseB"); _sB.__enter__()
    def _seg_b(k):
        sega = segas[k % 2]

        def _seg_i(i, carry):
            sl = pl.ds(i * 16, 16)
            t = etyp[sl] * _NP
            segd[sl] = t + ebuf[1, sl]
            sega[sl] = t + ebuf[0, sl]
            return carry
        lax.fori_loop(0, _SUBE // 16, _seg_i, None)
        return sega

    def _recip(k):
        cnt, val = cnts[k % 2], vals[k % 2]

        def _w_i(i, carry):
            sl = pl.ds(i * 16, 16)
            val[sl] = 1.0 / cnt[sl]
            return carry
        lax.fori_loop(0, _SUBE // 16, _w_i, None)
        return val

    def _phase_b(kchunks):
        ks = list(kchunks)
        pend_ld = _loads(ks[0])
        for c in pend_ld:
            c.wait()
        sega = _seg_b(ks[0])
        pend_g = pltpu.async_copy(counts_sh.at[segd], cnts[0], sem_g)
        pend_sc = None
        for j, k in enumerate(ks):
            if j + 1 < len(ks):
                for c in _loads(ks[j + 1]):
                    c.wait()
            pend_g.wait()
            if j + 1 < len(ks):
                next_sega = _seg_b(ks[j + 1])
                pend_g = pltpu.async_copy(counts_sh.at[segd],
                                          cnts[(j + 1) % 2], sem_g)
            val = _recip(j)
            if pend_sc is not None:
                pend_sc.wait()
            pend_sc = pltpu.async_copy(val, at_sh.at[sega], sem_sc, add=True)
            if j + 1 < len(ks):
                sega = next_sega
        pend_sc.wait()

    @pl.when(cid == 0)
    def _():
        _phase_b((0, 1))

    @pl.when(cid == 1)
    def _():
        _phase_b((2, 3))

    @pl.when((tid < _NMINI) & (cid == 1))
    def _():
        ebm = _EMAIN + tid * 128
        pltpu.sync_copy(edge_hbm.at[:, pl.ds(ebm, 128)], ebmini)
        pltpu.sync_copy(etyp_hbm.at[pl.ds(ebm, 128)], etmini)

        def _mi(i, carry):
            sl = pl.ds(i * 16, 16)
            t = etmini[sl] * _NP
            dmini[sl] = t + ebmini[1, sl]
            smini[sl] = t + ebmini[0, sl]
            return carry
        lax.fori_loop(0, 8, _mi, None)
        pltpu.async_copy(counts_sh.at[dmini], cmini, sem_g).wait()

        def _wm(i, carry):
            sl = pl.ds(i * 16, 16)
            vmini[sl] = 1.0 / cmini[sl]
            return carry
        lax.fori_loop(0, 8, _wm, None)
        pltpu.sync_copy(vmini, at_sh.at[smini], add=True)

    _sB.__exit__(None, None, None)
    plsc.subcore_barrier()

    def _writeout(a_out):
        pend = None
        for i in range(5):
            off = tid * _SLAB + i * _SUBE
            buf = cnts[i % 2]
            pltpu.sync_copy(at_sh.at[pl.ds(off, _SUBE)], buf)
            if pend is not None:
                pend.wait()
            pend = pltpu.async_copy(buf, a_out.at[pl.ds(off, _SUBE)], sem_sc)
        pend.wait()

    @pl.when(cid == 0)
    def _():
        _writeout(a_out0)

    @pl.when(cid == 1)
    def _():
        _writeout(a_out1)


def _tc_body(x_ref, at_ref, compt_ref, bases_ref, root_ref, bias_ref, o_ref):
    x = x_ref[...]
    s = jnp.dot(at_ref[...], x,
                preferred_element_type=jnp.float32)
    t = jnp.dot(compt_ref[...], s,
                preferred_element_type=jnp.float32)
    agg = jnp.zeros((1, _D), jnp.float32)
    for b in range(_B):
        agg = agg + jnp.dot(t[b:b + 1, :], bases_ref[b],
                            preferred_element_type=jnp.float32)
    xs = jnp.sum(x[:_N], axis=0, keepdims=True)
    g = agg + jnp.dot(xs, root_ref[...],
                      preferred_element_type=jnp.float32)
    g = g + float(_N) * bias_ref[...]
    nrm = jnp.sqrt(jnp.sum(g * g))
    o_ref[...] = g / jnp.maximum(nrm, 1e-5)


_tc_readout = pl.pallas_call(
    _tc_body,
    out_shape=jax.ShapeDtypeStruct((1, _D), jnp.float32),
)


def kernel(node_final_id, edge_index, edge_type, embedding, comp, bases, root, bias):
    a0, a1, x_pad = _sc_graph(edge_index.astype(jnp.int32),
                              edge_type.astype(jnp.int32),
                              node_final_id.astype(jnp.int32),
                              embedding)
    a_t = (a0 + a1).reshape(_R, _NP)
    return _tc_readout(x_pad, a_t, comp.T, bases, root, bias.reshape(1, _D))


# speedup vs baseline: 40.9391x; 1.0014x over previous
import functools

import jax
import jax.numpy as jnp
from jax import lax
from jax.experimental import pallas as pl
from jax.experimental.pallas import tpu as pltpu
from jax.experimental.pallas import tpu_sc as plsc

_N = 10000
_NP = 10240
_E = 320000
_D = 128
_R = 39
_B = 8

_NT = 16
_SUBE = 4992
_NSUB = 4
_EPT = _SUBE * _NSUB
_EMAIN = _EPT * _NT
_NMINI = 4
_CAP = _R * _NP
_SLAB = _CAP // _NT

_XPW = 320

_mesh = plsc.VectorSubcoreMesh(
    core_axis_name="c", subcore_axis_name="s", num_cores=2, num_subcores=_NT
)


@functools.partial(
    pl.kernel,
    out_type=(
        jax.ShapeDtypeStruct((_CAP,), jnp.float32),
        jax.ShapeDtypeStruct((_CAP,), jnp.float32),
        jax.ShapeDtypeStruct((_NP, _D), jnp.float32),
    ),
    mesh=_mesh,
    scratch_types=[
        pltpu.VMEM((2, _SUBE), jnp.int32),
        pltpu.VMEM((_SUBE,), jnp.int32),
        pltpu.VMEM((_SUBE,), jnp.int32),
        pltpu.VMEM((_SUBE,), jnp.int32),
        pltpu.VMEM((_SUBE,), jnp.int32),
        pltpu.VMEM((_SUBE,), jnp.float32),
        pltpu.VMEM((_SUBE,), jnp.float32),
        pltpu.VMEM((_SUBE,), jnp.float32),
        pltpu.VMEM((_SUBE,), jnp.float32),
        pltpu.VMEM((2, 128), jnp.int32),
        pltpu.VMEM((128,), jnp.int32),
        pltpu.VMEM((128,), jnp.int32),
        pltpu.VMEM((128,), jnp.int32),
        pltpu.VMEM((128,), jnp.float32),
        pltpu.VMEM((128,), jnp.float32),
        pltpu.VMEM((320,), jnp.int32),
        pltpu.VMEM((64, _D), jnp.float32),
        pltpu.VMEM((64, _D), jnp.float32),
        pltpu.VMEM_SHARED((_CAP,), jnp.float32),
        pltpu.VMEM_SHARED((_CAP,), jnp.float32),
        pltpu.SemaphoreType.DMA,
        pltpu.SemaphoreType.DMA,
        pltpu.SemaphoreType.DMA,
        pltpu.SemaphoreType.DMA,
    ],
)
def _sc_graph(edge_hbm, etyp_hbm, xids_hbm, emb_hbm,
              a_out0, a_out1, x_out,
              ebuf, etyp, segd, sega0, sega1, val0, val1, cnt0, cnt1,
              ebmini, etmini, smini, dmini, vmini, cmini,
              xidx, xrows0, xrows1, counts_sh, at_sh, sem_ld, sem_sc, sem_g,
              sem_xg):
    cid = lax.axis_index("c")
    tid = lax.axis_index("s")
    wid = tid * 2 + cid
    segas = (sega0, sega1)
    vals = (val0, val1)
    cnts = (cnt0, cnt1)
    xrows = (xrows0, xrows1)

    scope = jax.named_scope
    def _zero_i(i, carry):
        val0[pl.ds(i * 16, 16)] = jnp.zeros((16,), jnp.float32)
        return carry
    lax.fori_loop(0, _SUBE // 16, _zero_i, None)
    zcopies = []
    for shared in (counts_sh, at_sh):
        for k in range(5):
            zcopies.append(pltpu.async_copy(
                val0, shared.at[pl.ds(tid * _SLAB + k * _SUBE, _SUBE)],
                sem_sc))

    @pl.when(wid < 31)
    def _():
        pltpu.sync_copy(xids_hbm.at[pl.ds(wid * _XPW, _XPW)], xidx)

    @pl.when(wid == 31)
    def _():
        pltpu.sync_copy(xids_hbm.at[pl.ds(9920, 80)], xidx.at[pl.ds(0, 80)])
        pltpu.sync_copy(xids_hbm.at[pl.ds(0, 240)], xidx.at[pl.ds(80, 240)])

    def _xg_fire(j):
        return pltpu.async_copy(emb_hbm.at[xidx.at[pl.ds(j * 64, 64)]],
                                xrows[j % 2], sem_xg)

    def _xg_write(j):
        pltpu.sync_copy(xrows[j % 2], x_out.at[pl.ds(wid * _XPW + j * 64, 64)])

    pend_xg = _xg_fire(0)

    for c in zcopies:
        c.wait()

    def _ones_i(i, carry):
        val0[pl.ds(i * 16, 16)] = jnp.ones((16,), jnp.float32)
        val1[pl.ds(i * 16, 16)] = jnp.ones((16,), jnp.float32)
        return carry
    lax.fori_loop(0, _SUBE // 16, _ones_i, None)

    def _ones_m(i, carry):
        vmini[pl.ds(i * 16, 16)] = jnp.ones((16,), jnp.float32)
        return carry
    lax.fori_loop(0, 8, _ones_m, None)

    plsc.subcore_barrier()

    _sA = scope("phaseA"); _sA.__enter__()
    def _loads(k):
        eb = tid * _EPT + k * _SUBE
        la = pltpu.async_copy(edge_hbm.at[:, pl.ds(eb, _SUBE)], ebuf, sem_ld)
        lb = pltpu.async_copy(etyp_hbm.at[pl.ds(eb, _SUBE)], etyp, sem_ld)
        return la, lb

    def _seg_a(k):
        sega = segas[k % 2]

        def _seg_i(i, carry):
            sl = pl.ds(i * 16, 16)
            sega[sl] = etyp[sl] * _NP + ebuf[1, sl]
            return carry
        lax.fori_loop(0, _SUBE // 16, _seg_i, None)
        return sega

    pend_ld = _loads(0)
    pend_sc = None
    for k in range(_NSUB):
        for c in pend_ld:
            c.wait()
        sega = _seg_a(k)
        if k + 1 < _NSUB:
            pend_ld = _loads(k + 1)
        if pend_sc is not None:
            pend_sc.wait()
        pend_sc = pltpu.async_copy(vals[k % 2], counts_sh.at[sega], sem_sc,
                                   add=True)
        pend_xg.wait()
        nxt = _xg_fire(k + 1)
        _xg_write(k)
        pend_xg = nxt
    pend_sc.wait()
    pend_xg.wait()
    _xg_write(4)

    @pl.when(tid < _NMINI)
    def _():
        ebm = _EMAIN + tid * 128
        pltpu.sync_copy(edge_hbm.at[:, pl.ds(ebm, 128)], ebmini)
        pltpu.sync_copy(etyp_hbm.at[pl.ds(ebm, 128)], etmini)

        def _mi(i, carry):
            sl = pl.ds(i * 16, 16)
            smini[sl] = etmini[sl] * _NP + ebmini[1, sl]
            return carry
        lax.fori_loop(0, 8, _mi, None)
        pltpu.sync_copy(vmini, counts_sh.at[smini], add=True)

    _sA.__exit__(None, None, None)
    plsc.subcore_barrier()

    _sB = scope("phaseB"); _sB.__enter__()
    def _seg_b(k):
        sega = segas[k % 2]

        def _seg_i(i, carry):
            sl = pl.ds(i * 16, 16)
            t = etyp[sl] * _NP
            segd[sl] = t + ebuf[1, sl]
            sega[sl] = t + ebuf[0, sl]
            return carry
        lax.fori_loop(0, _SUBE // 16, _seg_i, None)
        return sega

    def _recip(k):
        cnt, val = cnts[k % 2], vals[k % 2]

        def _w_i(i, carry):
            sl = pl.ds(i * 16, 16)
            val[sl] = 1.0 / cnt[sl]
            return carry
        lax.fori_loop(0, _SUBE // 16, _w_i, None)
        return val

    def _phase_b(kchunks):
        ks = list(kchunks)
        pend_ld = _loads(ks[0])
        for c in pend_ld:
            c.wait()
        sega = _seg_b(ks[0])
        pend_g = pltpu.async_copy(counts_sh.at[segd], cnts[0], sem_g)
        pend_sc = None
        for j, k in enumerate(ks):
            if j + 1 < len(ks):
                for c in _loads(ks[j + 1]):
                    c.wait()
            pend_g.wait()
            if j + 1 < len(ks):
                next_sega = _seg_b(ks[j + 1])
                pend_g = pltpu.async_copy(counts_sh.at[segd],
                                          cnts[(j + 1) % 2], sem_g)
            val = _recip(j)
            if pend_sc is not None:
                pend_sc.wait()
            pend_sc = pltpu.async_copy(val, at_sh.at[sega], sem_sc, add=True)
            if j + 1 < len(ks):
                sega = next_sega
        pend_sc.wait()

    @pl.when(cid == 0)
    def _():
        _phase_b((0, 1))

    @pl.when(cid == 1)
    def _():
        _phase_b((2, 3))

    @pl.when((tid < _NMINI) & (cid == 1))
    def _():
        ebm = _EMAIN + tid * 128
        pltpu.sync_copy(edge_hbm.at[:, pl.ds(ebm, 128)], ebmini)
        pltpu.sync_copy(etyp_hbm.at[pl.ds(ebm, 128)], etmini)

        def _mi(i, carry):
            sl = pl.ds(i * 16, 16)
            t = etmini[sl] * _NP
            dmini[sl] = t + ebmini[1, sl]
            smini[sl] = t + ebmini[0, sl]
            return carry
        lax.fori_loop(0, 8, _mi, None)
        pltpu.async_copy(counts_sh.at[dmini], cmini, sem_g).wait()

        def _wm(i, carry):
            sl = pl.ds(i * 16, 16)
            vmini[sl] = 1.0 / cmini[sl]
            return carry
        lax.fori_loop(0, 8, _wm, None)
        pltpu.sync_copy(vmini, at_sh.at[smini], add=True)

    _sB.__exit__(None, None, None)
    plsc.subcore_barrier()

    def _writeout(a_out):
        pend = None
        for i in range(5):
            off = tid * _SLAB + i * _SUBE
            buf = cnts[i % 2]
            pltpu.sync_copy(at_sh.at[pl.ds(off, _SUBE)], buf)
            if pend is not None:
                pend.wait()
            pend = pltpu.async_copy(buf, a_out.at[pl.ds(off, _SUBE)], sem_sc)
        pend.wait()

    @pl.when(cid == 0)
    def _():
        _writeout(a_out0)

    @pl.when(cid == 1)
    def _():
        _writeout(a_out1)


def _tc_body(x_ref, at_ref, compt_ref, bases_ref, root_ref, bias_ref, o_ref):
    x = x_ref[...]
    s = jnp.dot(at_ref[...], x,
                preferred_element_type=jnp.float32)
    t = jnp.dot(compt_ref[...], s,
                preferred_element_type=jnp.float32)
    agg = jnp.zeros((1, _D), jnp.float32)
    for b in range(_B):
        agg = agg + jnp.dot(t[b:b + 1, :], bases_ref[b],
                            preferred_element_type=jnp.float32)
    xs = jnp.sum(x[:_N], axis=0, keepdims=True)
    g = agg + jnp.dot(xs, root_ref[...],
                      preferred_element_type=jnp.float32)
    g = g + float(_N) * bias_ref[...]
    nrm = jnp.sqrt(jnp.sum(g * g))
    o_ref[...] = g / jnp.maximum(nrm, 1e-5)


_tc_readout = pl.pallas_call(
    _tc_body,
    out_shape=jax.ShapeDtypeStruct((1, _D), jnp.float32),
)


def kernel(node_final_id, edge_index, edge_type, embedding, comp, bases, root, bias):
    a0, a1, x_pad = _sc_graph(edge_index.astype(jnp.int32),
                              edge_type.astype(jnp.int32),
                              node_final_id.astype(jnp.int32),
                              embedding)
    a_t = a0.reshape(_R, _NP) + a1.reshape(_R, _NP)
    return _tc_readout(x_pad, a_t, comp.T, bases, root, bias.reshape(1, _D))
